# Initial kernel scaffold; baseline (speedup 1.0000x reference)
import jax
import jax.numpy as jnp
from jax.experimental import pallas as pl


def kernel(features, xyz, fc1_w, fc1_b, fc2_w, fc2_b, fd1_w, fd1_b, fd2_w, fd2_b, wq, wk, wv):
    raise NotImplementedError("write your pallas kernel here")


# speedup vs baseline: 6.6111x; 6.6111x over previous
import functools

import jax
import jax.numpy as jnp
from jax import lax
from jax.experimental import pallas as pl
from jax.experimental.pallas import tpu as pltpu

F32 = jnp.float32
BF16 = jnp.bfloat16
N = 2048
RB = 256
K_SP = 16
K_TMP = 8
NEG_INF = float("-inf")
POS_INF = float("inf")


def _bf(a):
    return a.astype(BF16)


def _dotb(a, b, dims=((1,), (0,))):
    return lax.dot_general(_bf(a), _bf(b), (dims, ((), ())),
                           preferred_element_type=F32)


def _split3(a):
    hi = a.astype(BF16)
    r1 = a - hi.astype(F32)
    mid = r1.astype(BF16)
    lo = (r1 - mid.astype(F32)).astype(BF16)
    return hi, mid, lo


def _gather3(ohb, parts):
    h, m, l = parts
    g = lambda p: lax.dot_general(ohb, p, ((((1,), (0,))), ((), ())),
                                  preferred_element_type=F32)
    return (g(h) + g(m)) + g(l)


def _proj_kernel(feat_ref, fc1_w, fc1_b, wq, wk, wv, xq_o, xk_o, xv_o):
    x = _dotb(feat_ref[0], fc1_w[...]) + fc1_b[...]
    xq_o[0] = _dotb(x, wq[...])
    xk_o[0] = _dotb(x, wk[...])
    xv_o[0] = _dotb(x, wv[...])


def _argmin_mask(d, iota):
    minv = jnp.min(d, axis=1, keepdims=True)
    idx = jnp.min(jnp.where(d == minv, iota, N), axis=1, keepdims=True)
    return iota == idx


def _spatial_kernel(xq_ref, xk_ref, xv_ref, pts_ref,
                    fd1_w, fd1_b, fd2_w, fd2_b, sp_o):
    r = pl.program_id(1)
    q = xq_ref[0]
    xk3 = _split3(xk_ref[0])
    xv3 = _split3(xv_ref[0])
    pts = pts_ref[0]
    pts3 = _split3(pts)
    pts_blk = pts_ref[0, pl.ds(r * RB, RB), :]

    sq_all = jnp.sum(pts * pts, axis=1)
    sq_blk = jnp.sum(pts_blk * pts_blk, axis=1, keepdims=True)
    pp = _dotb(pts_blk, pts, ((1,), (1,)))
    d = sq_blk + sq_all[None, :] - 2.0 * pp

    fd2 = fd2_w[...]
    fd2_hi, fd2_mid, fd2_lo = _split3(fd2)
    iota = lax.broadcasted_iota(jnp.int32, (RB, N), 1)
    lj, vj = [], []
    for j in range(K_SP):
        oh = _argmin_mask(d, iota)
        d = jnp.where(oh, POS_INF, d)
        ohb = oh.astype(BF16)
        kxyz = _gather3(ohb, pts3)
        delta = pts_blk - kxyz
        h1 = (delta[:, 0:1] * fd1_w[0:1, :]
              + delta[:, 1:2] * fd1_w[1:2, :]
              + delta[:, 2:3] * fd1_w[2:3, :]) + fd1_b[...]
        h1 = jnp.maximum(h1, 0.0)
        h1b = _bf(h1)
        g2 = lambda w: lax.dot_general(h1b, w, ((((1,), (0,))), ((), ())),
                                       preferred_element_type=F32)
        pos = ((g2(fd2_hi) + g2(fd2_mid)) + g2(fd2_lo)) + fd2_b[...]
        kj = _gather3(ohb, xk3) + pos
        vj.append(_gather3(ohb, xv3) + pos)
        lj.append(jnp.sum(q * kj, axis=1, keepdims=True) * 0.125)

    m = functools.reduce(jnp.maximum, lj)
    e = [jnp.exp(x - m) for x in lj]
    z = functools.reduce(jnp.add, e)
    acc = jnp.zeros((RB, 64), F32)
    for j in range(K_SP):
        acc = acc + (e[j] / z) * vj[j]
    sp_o[0] = acc


def _temporal_kernel(sp_ref, fc2_w, fc2_b, out_o):
    r = pl.program_id(1)
    sp = sp_ref[0]
    sp_blk = sp_ref[0, pl.ds(r * RB, RB), :]
    nrm = jnp.sqrt(jnp.sum(sp * sp, axis=1, keepdims=True))
    fn = sp / jnp.maximum(nrm, 1e-12)
    nrm_b = jnp.sqrt(jnp.sum(sp_blk * sp_blk, axis=1, keepdims=True))
    fn_blk = sp_blk / jnp.maximum(nrm_b, 1e-12)
    sim = _dotb(fn_blk, fn, ((1,), (1,)))
    s = _dotb(sp_blk, sp, ((1,), (1,)))

    iota = lax.broadcasted_iota(jnp.int32, (RB, N), 1)
    simw = sim
    for _ in range(K_TMP):
        maxv = jnp.max(simw, axis=1, keepdims=True)
        idx = jnp.min(jnp.where(simw == maxv, iota, N), axis=1, keepdims=True)
        simw = jnp.where(iota == idx, NEG_INF, simw)
    sel = simw == NEG_INF
    logits = jnp.where(sel, s * 0.125, NEG_INF)
    m = jnp.max(logits, axis=1, keepdims=True)
    e = jnp.exp(logits - m)
    at = e / jnp.sum(e, axis=1, keepdims=True)
    at_b = _bf(at)
    sp3 = _split3(sp)
    gt = lambda p: lax.dot_general(at_b, p, ((((1,), (0,))), ((), ())),
                                   preferred_element_type=F32)
    temporal = (gt(sp3[0]) + gt(sp3[1])) + gt(sp3[2])

    fused_b = _bf(jnp.concatenate([sp_blk, temporal], axis=1))
    fc23 = _split3(fc2_w[...])
    go = lambda w: lax.dot_general(w, fused_b, ((((0,), (1,))), ((), ())),
                                   preferred_element_type=F32)
    out_o[0] = ((go(fc23[0]) + go(fc23[1])) + go(fc23[2])) + fc2_b[...]


def kernel(features, xyz, fc1_w, fc1_b, fc2_w, fc2_b,
           fd1_w, fd1_b, fd2_w, fd2_b, wq, wk, wv):
    B, T, n, dp = features.shape
    F = B * T
    feat = features.reshape(F, n, dp)
    pts = xyz.reshape(F, n, 3)
    fc1_b2 = fc1_b.reshape(1, -1)
    fd1_b2 = fd1_b.reshape(1, -1)
    fd2_b2 = fd2_b.reshape(1, -1)
    fc2_b2 = fc2_b.reshape(-1, 1)

    wfull = lambda: pl.BlockSpec(index_map=lambda f: (0, 0))
    xq, xk, xv = pl.pallas_call(
        _proj_kernel,
        grid=(F,),
        in_specs=[pl.BlockSpec((1, n, dp), lambda f: (f, 0, 0)),
                  wfull(), wfull(), wfull(), wfull(), wfull()],
        out_specs=[pl.BlockSpec((1, n, 64), lambda f: (f, 0, 0))] * 3,
        out_shape=[jax.ShapeDtypeStruct((F, n, 64), F32)] * 3,
    )(feat, fc1_w, fc1_b2, wq, wk, wv)

    R = n // RB
    w2 = lambda: pl.BlockSpec(index_map=lambda f, r: (0, 0))
    spatial = pl.pallas_call(
        _spatial_kernel,
        grid=(F, R),
        in_specs=[pl.BlockSpec((1, RB, 64), lambda f, r: (f, r, 0)),
                  pl.BlockSpec((1, n, 64), lambda f, r: (f, 0, 0)),
                  pl.BlockSpec((1, n, 64), lambda f, r: (f, 0, 0)),
                  pl.BlockSpec((1, n, 3), lambda f, r: (f, 0, 0)),
                  w2(), w2(), w2(), w2()],
        out_specs=pl.BlockSpec((1, RB, 64), lambda f, r: (f, r, 0)),
        out_shape=jax.ShapeDtypeStruct((F, n, 64), F32),
    )(xq, xk, xv, pts, fd1_w, fd1_b2, fd2_w, fd2_b2)

    out = pl.pallas_call(
        _temporal_kernel,
        grid=(F, R),
        in_specs=[pl.BlockSpec((1, n, 64), lambda f, r: (f, 0, 0)),
                  w2(), w2()],
        out_specs=pl.BlockSpec((1, 64, RB), lambda f, r: (f, 0, r)),
        out_shape=jax.ShapeDtypeStruct((F, 64, n), F32),
    )(spatial, fc2_w, fc2_b2)

    return out.reshape(B, T, 64, n)


# speedup vs baseline: 10.4520x; 1.5810x over previous
import functools

import jax
import jax.numpy as jnp
from jax import lax
from jax.experimental import pallas as pl
from jax.experimental.pallas import tpu as pltpu

F32 = jnp.float32
BF16 = jnp.bfloat16
N = 2048
RB = 256
K_SP = 16
K_TMP = 8
NEG_INF = float("-inf")
POS_INF = float("inf")


def _bf(a):
    return a.astype(BF16)


def _dotb(a, b, dims=((1,), (0,))):
    return lax.dot_general(_bf(a), _bf(b), (dims, ((), ())),
                           preferred_element_type=F32)


def _split3(a):
    hi = a.astype(BF16)
    r1 = a - hi.astype(F32)
    mid = r1.astype(BF16)
    lo = (r1 - mid.astype(F32)).astype(BF16)
    return hi, mid, lo


def _gather3(ohb, parts):
    h, m, l = parts
    g = lambda p: lax.dot_general(ohb, p, ((((1,), (0,))), ((), ())),
                                  preferred_element_type=F32)
    return (g(h) + g(m)) + g(l)


def _proj_kernel(feat_ref, fc1_w, fc1_b, wq, wk, wv, xq_o, xk_o, xv_o):
    x = _dotb(feat_ref[0], fc1_w[...]) + fc1_b[...]
    xq_o[0] = _dotb(x, wq[...])
    xk_o[0] = _dotb(x, wk[...])
    xv_o[0] = _dotb(x, wv[...])


def _argmin_mask(d, iota):
    idx = jnp.argmin(d, axis=1)[:, None]
    return iota == idx


def _spatial_kernel(xq_ref, xk_ref, xv_ref, pts_ref,
                    fd1_w, fd1_b, fd2_w, fd2_b, sp_o):
    r = pl.program_id(1)
    q = xq_ref[0]
    xk3 = _split3(xk_ref[0])
    xv3 = _split3(xv_ref[0])
    pts = pts_ref[0]
    pts3 = _split3(pts)
    pts_blk = pts_ref[0, pl.ds(r * RB, RB), :]
    table = jnp.concatenate(list(xk3) + list(xv3) + list(pts3), axis=1)

    sq_all = jnp.sum(pts * pts, axis=1)
    sq_blk = jnp.sum(pts_blk * pts_blk, axis=1, keepdims=True)
    pp = _dotb(pts_blk, pts, ((1,), (1,)))
    d = sq_blk + sq_all[None, :] - 2.0 * pp

    fd2 = fd2_w[...]
    fd2_hi, fd2_mid, fd2_lo = _split3(fd2)
    iota = lax.broadcasted_iota(jnp.int32, (RB, N), 1)
    lj, vj = [], []
    for j in range(K_SP):
        oh = _argmin_mask(d, iota)
        d = jnp.where(oh, POS_INF, d)
        ohb = oh.astype(BF16)
        g = lax.dot_general(ohb, table, ((((1,), (0,))), ((), ())),
                            preferred_element_type=F32)
        kg = (g[:, 0:64] + g[:, 64:128]) + g[:, 128:192]
        vg = (g[:, 192:256] + g[:, 256:320]) + g[:, 320:384]
        kxyz = (g[:, 384:387] + g[:, 387:390]) + g[:, 390:393]
        delta = pts_blk - kxyz
        h1 = (delta[:, 0:1] * fd1_w[0:1, :]
              + delta[:, 1:2] * fd1_w[1:2, :]
              + delta[:, 2:3] * fd1_w[2:3, :]) + fd1_b[...]
        h1 = jnp.maximum(h1, 0.0)
        h1b = _bf(h1)
        g2 = lambda w: lax.dot_general(h1b, w, ((((1,), (0,))), ((), ())),
                                       preferred_element_type=F32)
        pos = ((g2(fd2_hi) + g2(fd2_mid)) + g2(fd2_lo)) + fd2_b[...]
        kj = kg + pos
        vj.append(vg + pos)
        lj.append(jnp.sum(q * kj, axis=1, keepdims=True) * 0.125)

    m = functools.reduce(jnp.maximum, lj)
    e = [jnp.exp(x - m) for x in lj]
    z = functools.reduce(jnp.add, e)
    acc = jnp.zeros((RB, 64), F32)
    for j in range(K_SP):
        acc = acc + (e[j] / z) * vj[j]
    sp_o[0] = acc


def _temporal_kernel(sp_ref, fc2_w, fc2_b, out_o):
    r = pl.program_id(1)
    sp = sp_ref[0]
    sp_blk = sp_ref[0, pl.ds(r * RB, RB), :]
    nrm = jnp.sqrt(jnp.sum(sp * sp, axis=1, keepdims=True))
    fn = sp / jnp.maximum(nrm, 1e-12)
    nrm_b = jnp.sqrt(jnp.sum(sp_blk * sp_blk, axis=1, keepdims=True))
    fn_blk = sp_blk / jnp.maximum(nrm_b, 1e-12)
    sim = _dotb(fn_blk, fn, ((1,), (1,)))
    s = _dotb(sp_blk, sp, ((1,), (1,)))

    iota = lax.broadcasted_iota(jnp.int32, (RB, N), 1)
    simw = sim
    for _ in range(K_TMP):
        idx = jnp.argmax(simw, axis=1)[:, None]
        simw = jnp.where(iota == idx, NEG_INF, simw)
    sel = simw == NEG_INF
    logits = jnp.where(sel, s * 0.125, NEG_INF)
    m = jnp.max(logits, axis=1, keepdims=True)
    e = jnp.exp(logits - m)
    at = e / jnp.sum(e, axis=1, keepdims=True)
    at_b = _bf(at)
    sp3 = _split3(sp)
    gt = lambda p: lax.dot_general(at_b, p, ((((1,), (0,))), ((), ())),
                                   preferred_element_type=F32)
    temporal = (gt(sp3[0]) + gt(sp3[1])) + gt(sp3[2])

    fused_b = _bf(jnp.concatenate([sp_blk, temporal], axis=1))
    fc23 = _split3(fc2_w[...])
    go = lambda w: lax.dot_general(w, fused_b, ((((0,), (1,))), ((), ())),
                                   preferred_element_type=F32)
    out_o[0] = ((go(fc23[0]) + go(fc23[1])) + go(fc23[2])) + fc2_b[...]


def kernel(features, xyz, fc1_w, fc1_b, fc2_w, fc2_b,
           fd1_w, fd1_b, fd2_w, fd2_b, wq, wk, wv):
    B, T, n, dp = features.shape
    F = B * T
    feat = features.reshape(F, n, dp)
    pts = xyz.reshape(F, n, 3)
    fc1_b2 = fc1_b.reshape(1, -1)
    fd1_b2 = fd1_b.reshape(1, -1)
    fd2_b2 = fd2_b.reshape(1, -1)
    fc2_b2 = fc2_b.reshape(-1, 1)

    wfull = lambda: pl.BlockSpec(index_map=lambda f: (0, 0))
    xq, xk, xv = pl.pallas_call(
        _proj_kernel,
        grid=(F,),
        in_specs=[pl.BlockSpec((1, n, dp), lambda f: (f, 0, 0)),
                  wfull(), wfull(), wfull(), wfull(), wfull()],
        out_specs=[pl.BlockSpec((1, n, 64), lambda f: (f, 0, 0))] * 3,
        out_shape=[jax.ShapeDtypeStruct((F, n, 64), F32)] * 3,
    )(feat, fc1_w, fc1_b2, wq, wk, wv)

    R = n // RB
    w2 = lambda: pl.BlockSpec(index_map=lambda f, r: (0, 0))
    spatial = pl.pallas_call(
        _spatial_kernel,
        grid=(F, R),
        in_specs=[pl.BlockSpec((1, RB, 64), lambda f, r: (f, r, 0)),
                  pl.BlockSpec((1, n, 64), lambda f, r: (f, 0, 0)),
                  pl.BlockSpec((1, n, 64), lambda f, r: (f, 0, 0)),
                  pl.BlockSpec((1, n, 3), lambda f, r: (f, 0, 0)),
                  w2(), w2(), w2(), w2()],
        out_specs=pl.BlockSpec((1, RB, 64), lambda f, r: (f, r, 0)),
        out_shape=jax.ShapeDtypeStruct((F, n, 64), F32),
    )(xq, xk, xv, pts, fd1_w, fd1_b2, fd2_w, fd2_b2)

    out = pl.pallas_call(
        _temporal_kernel,
        grid=(F, R),
        in_specs=[pl.BlockSpec((1, n, 64), lambda f, r: (f, 0, 0)),
                  w2(), w2()],
        out_specs=pl.BlockSpec((1, 64, RB), lambda f, r: (f, 0, r)),
        out_shape=jax.ShapeDtypeStruct((F, 64, n), F32),
    )(spatial, fc2_w, fc2_b2)

    return out.reshape(B, T, 64, n)


# speedup vs baseline: 11.8733x; 1.1360x over previous
import functools

import jax
import jax.numpy as jnp
from jax import lax
from jax.experimental import pallas as pl
from jax.experimental.pallas import tpu as pltpu

F32 = jnp.float32
BF16 = jnp.bfloat16
N = 2048
RB = 256
K_SP = 16
K_TMP = 8
NEG_INF = float("-inf")
POS_INF = float("inf")


def _bf(a):
    return a.astype(BF16)


def _dotb(a, b, dims=((1,), (0,))):
    return lax.dot_general(_bf(a), _bf(b), (dims, ((), ())),
                           preferred_element_type=F32)


def _split3(a):
    hi = a.astype(BF16)
    r1 = a - hi.astype(F32)
    mid = r1.astype(BF16)
    lo = (r1 - mid.astype(F32)).astype(BF16)
    return hi, mid, lo


def _gather3(ohb, parts):
    h, m, l = parts
    g = lambda p: lax.dot_general(ohb, p, ((((1,), (0,))), ((), ())),
                                  preferred_element_type=F32)
    return (g(h) + g(m)) + g(l)


def _proj_kernel(feat_ref, fc1_w, fc1_b, wq, wk, wv, xq_o, xk_o, xv_o):
    x = _dotb(feat_ref[0], fc1_w[...]) + fc1_b[...]
    xq_o[0] = _dotb(x, wq[...])
    xk_o[0] = _dotb(x, wk[...])
    xv_o[0] = _dotb(x, wv[...])


def _argmin_mask(d, iota):
    idx = jnp.argmin(d, axis=1)[:, None]
    return iota == idx


def _spatial_kernel(xq_ref, xk_ref, xv_ref, pts_ref,
                    fd1_w, fd1_b, fd2_w, fd2_b, sp_o):
    r = pl.program_id(1)
    q = xq_ref[0]
    xk3 = _split3(xk_ref[0])
    xv3 = _split3(xv_ref[0])
    pts = pts_ref[0]
    pts3 = _split3(pts)
    pts_blk = pts_ref[0, pl.ds(r * RB, RB), :]
    table = jnp.concatenate(list(xk3) + list(xv3) + list(pts3), axis=1)

    sq_all = jnp.sum(pts * pts, axis=1)
    sq_blk = jnp.sum(pts_blk * pts_blk, axis=1, keepdims=True)
    pp = _dotb(pts_blk, pts, ((1,), (1,)))
    d = sq_blk + sq_all[None, :] - 2.0 * pp

    fd2 = fd2_w[...]
    fd2_hi, fd2_mid, fd2_lo = _split3(fd2)
    iota = lax.broadcasted_iota(jnp.int32, (RB, N), 1)
    idx_list = []
    for j in range(K_SP):
        idx = jnp.argmin(d, axis=1)[:, None]
        d = jnp.where(iota == idx, POS_INF, d)
        idx_list.append(idx)
    idx_all = jnp.concatenate(idx_list, axis=0)

    M = K_SP * RB
    iota2 = lax.broadcasted_iota(jnp.int32, (M, N), 1)
    ohb = (iota2 == idx_all).astype(BF16)
    g = lax.dot_general(ohb, table, ((((1,), (0,))), ((), ())),
                        preferred_element_type=F32)
    kg = (g[:, 0:64] + g[:, 64:128]) + g[:, 128:192]
    vg = (g[:, 192:256] + g[:, 256:320]) + g[:, 320:384]
    kxyz = (g[:, 384:387] + g[:, 387:390]) + g[:, 390:393]
    pts_rep = jnp.concatenate([pts_blk] * K_SP, axis=0)
    delta = pts_rep - kxyz
    h1 = (delta[:, 0:1] * fd1_w[0:1, :]
          + delta[:, 1:2] * fd1_w[1:2, :]
          + delta[:, 2:3] * fd1_w[2:3, :]) + fd1_b[...]
    h1b = _bf(jnp.maximum(h1, 0.0))
    g2 = lambda w: lax.dot_general(h1b, w, ((((1,), (0,))), ((), ())),
                                   preferred_element_type=F32)
    pos = ((g2(fd2_hi) + g2(fd2_mid)) + g2(fd2_lo)) + fd2_b[...]
    kj = kg + pos
    vj = vg + pos
    q_rep = jnp.concatenate([q] * K_SP, axis=0)
    l_all = jnp.sum(q_rep * kj, axis=1, keepdims=True) * 0.125

    lj = [l_all[j * RB:(j + 1) * RB] for j in range(K_SP)]
    m = functools.reduce(jnp.maximum, lj)
    e = [jnp.exp(x - m) for x in lj]
    z = functools.reduce(jnp.add, e)
    acc = jnp.zeros((RB, 64), F32)
    for j in range(K_SP):
        acc = acc + (e[j] / z) * vj[j * RB:(j + 1) * RB]
    sp_o[0] = acc


def _temporal_kernel(sp_ref, fc2_w, fc2_b, out_o):
    r = pl.program_id(1)
    sp = sp_ref[0]
    sp_blk = sp_ref[0, pl.ds(r * RB, RB), :]
    nrm = jnp.sqrt(jnp.sum(sp * sp, axis=1, keepdims=True))
    fn = sp / jnp.maximum(nrm, 1e-12)
    nrm_b = jnp.sqrt(jnp.sum(sp_blk * sp_blk, axis=1, keepdims=True))
    fn_blk = sp_blk / jnp.maximum(nrm_b, 1e-12)
    sim = _dotb(fn_blk, fn, ((1,), (1,)))
    s = _dotb(sp_blk, sp, ((1,), (1,)))

    iota = lax.broadcasted_iota(jnp.int32, (RB, N), 1)
    simw = sim
    for _ in range(K_TMP):
        idx = jnp.argmax(simw, axis=1)[:, None]
        simw = jnp.where(iota == idx, NEG_INF, simw)
    sel = simw == NEG_INF
    logits = jnp.where(sel, s * 0.125, NEG_INF)
    m = jnp.max(logits, axis=1, keepdims=True)
    e = jnp.exp(logits - m)
    at = e / jnp.sum(e, axis=1, keepdims=True)
    at_b = _bf(at)
    sp3 = _split3(sp)
    gt = lambda p: lax.dot_general(at_b, p, ((((1,), (0,))), ((), ())),
                                   preferred_element_type=F32)
    temporal = (gt(sp3[0]) + gt(sp3[1])) + gt(sp3[2])

    fused_b = _bf(jnp.concatenate([sp_blk, temporal], axis=1))
    fc23 = _split3(fc2_w[...])
    go = lambda w: lax.dot_general(w, fused_b, ((((0,), (1,))), ((), ())),
                                   preferred_element_type=F32)
    out_o[0] = ((go(fc23[0]) + go(fc23[1])) + go(fc23[2])) + fc2_b[...]


def kernel(features, xyz, fc1_w, fc1_b, fc2_w, fc2_b,
           fd1_w, fd1_b, fd2_w, fd2_b, wq, wk, wv):
    B, T, n, dp = features.shape
    F = B * T
    feat = features.reshape(F, n, dp)
    pts = xyz.reshape(F, n, 3)
    fc1_b2 = fc1_b.reshape(1, -1)
    fd1_b2 = fd1_b.reshape(1, -1)
    fd2_b2 = fd2_b.reshape(1, -1)
    fc2_b2 = fc2_b.reshape(-1, 1)

    wfull = lambda: pl.BlockSpec(index_map=lambda f: (0, 0))
    xq, xk, xv = pl.pallas_call(
        _proj_kernel,
        grid=(F,),
        in_specs=[pl.BlockSpec((1, n, dp), lambda f: (f, 0, 0)),
                  wfull(), wfull(), wfull(), wfull(), wfull()],
        out_specs=[pl.BlockSpec((1, n, 64), lambda f: (f, 0, 0))] * 3,
        out_shape=[jax.ShapeDtypeStruct((F, n, 64), F32)] * 3,
    )(feat, fc1_w, fc1_b2, wq, wk, wv)

    R = n // RB
    w2 = lambda: pl.BlockSpec(index_map=lambda f, r: (0, 0))
    spatial = pl.pallas_call(
        _spatial_kernel,
        grid=(F, R),
        in_specs=[pl.BlockSpec((1, RB, 64), lambda f, r: (f, r, 0)),
                  pl.BlockSpec((1, n, 64), lambda f, r: (f, 0, 0)),
                  pl.BlockSpec((1, n, 64), lambda f, r: (f, 0, 0)),
                  pl.BlockSpec((1, n, 3), lambda f, r: (f, 0, 0)),
                  w2(), w2(), w2(), w2()],
        out_specs=pl.BlockSpec((1, RB, 64), lambda f, r: (f, r, 0)),
        out_shape=jax.ShapeDtypeStruct((F, n, 64), F32),
    )(xq, xk, xv, pts, fd1_w, fd1_b2, fd2_w, fd2_b2)

    out = pl.pallas_call(
        _temporal_kernel,
        grid=(F, R),
        in_specs=[pl.BlockSpec((1, n, 64), lambda f, r: (f, 0, 0)),
                  w2(), w2()],
        out_specs=pl.BlockSpec((1, 64, RB), lambda f, r: (f, 0, r)),
        out_shape=jax.ShapeDtypeStruct((F, 64, n), F32),
    )(spatial, fc2_w, fc2_b2)

    return out.reshape(B, T, 64, n)
